# baseline (speedup 1.0000x reference)
import functools

import jax
import jax.numpy as jnp
from jax import lax
from jax.experimental import pallas as pl
from jax.experimental.pallas import tpu as pltpu
from jax.experimental.pallas import tpu_sc as plsc

NUM_COLORS = 10
D_MODEL = 128
MAX_LEN = 900
B = 1024
L = 900

NUM_WORKERS = 32
ROWS_W = B // NUM_WORKERS
LANES = 16
LROW = 912
NVEC = LROW // LANES


def _comb_body(tok_ref, pos_ref, out_ref):
    out_ref[...] = pos_ref[...][:, None, :] + tok_ref[...][None, :, :]


def _build_comb(tok, pos):
    return pl.pallas_call(
        _comb_body,
        out_shape=jax.ShapeDtypeStruct((MAX_LEN, NUM_COLORS, D_MODEL),
                                       jnp.float32),
    )(tok, pos)


_sc_mesh = plsc.VectorSubcoreMesh(core_axis_name="c", subcore_axis_name="s")


@functools.partial(
    pl.kernel,
    out_type=jax.ShapeDtypeStruct((B, L, D_MODEL), jnp.float32),
    mesh=_sc_mesh,
    scratch_types=[
        pltpu.VMEM((LROW,), jnp.int32),
        pltpu.VMEM((LROW,), jnp.int32),
        pltpu.VMEM((1024,), jnp.int32),
        pltpu.VMEM((1024,), jnp.int32),
        pltpu.VMEM((L, D_MODEL), jnp.float32),
        pltpu.SemaphoreType.DMA,
        pltpu.SemaphoreType.DMA,
    ],
)
def _sc_gather(comb_hbm, x_hbm, out_hbm, xv, l10v, idx0, idx1, rowsv,
               gsem, wsem):
    s = lax.axis_index("s")
    wid = s * 2 + lax.axis_index("c")
    row0 = wid * ROWS_W
    idxb = (idx0, idx1)
    lane = lax.iota(jnp.int32, LANES)

    def l10_body(k, c):
        o = k * LANES
        l10v[pl.ds(o, LANES)] = (o + lane) * NUM_COLORS
        return c

    lax.fori_loop(0, NVEC, l10_body, 0)

    def _row_of(t):
        return row0 + t

    def stage_x(t):
        pltpu.sync_copy(x_hbm.at[pl.ds(_row_of(t) * LROW, LROW)], xv)

    def idx_compute(p):
        def body(k, c):
            o = k * LANES
            idxb[p][pl.ds(o, LANES)] = l10v[pl.ds(o, LANES)] + \
                xv[pl.ds(o, LANES)]
            return c

        lax.fori_loop(0, NVEC, body, 0)

    def gfire(p):
        for j in range(7):
            pltpu.async_copy(
                comb_hbm.at[idxb[p].at[pl.ds(j * 120, 120)]],
                rowsv.at[pl.ds(j * 120, 120)], gsem)
        pltpu.async_copy(
            comb_hbm.at[idxb[p].at[pl.ds(840, 60)]],
            rowsv.at[pl.ds(840, 60)], gsem)

    def gdrain(p):
        for j in range(7):
            pltpu.make_async_copy(
                comb_hbm.at[idxb[p].at[pl.ds(j * 120, 120)]],
                rowsv.at[pl.ds(j * 120, 120)], gsem).wait()
        pltpu.make_async_copy(
            comb_hbm.at[idxb[p].at[pl.ds(840, 60)]],
            rowsv.at[pl.ds(840, 60)], gsem).wait()

    def wfire(t):
        pltpu.async_copy(rowsv, out_hbm.at[_row_of(t)], wsem)

    def wwait():
        pltpu.make_async_copy(rowsv, out_hbm.at[0], wsem).wait()

    stage_x(0)
    idx_compute(0)

    def step_body(k, c):
        for sub in range(2):
            t = k * 2 + sub
            p = sub
            if sub == 0:
                @pl.when(k > 0)
                def _():
                    wwait()
            else:
                wwait()

            gfire(p)

            if sub == 0:
                stage_x(t + 1)
                idx_compute(1 - p)
            else:
                @pl.when(k < ROWS_W // 2 - 1)
                def _(t=t, p=p):
                    stage_x(t + 1)
                    idx_compute(1 - p)

            gdrain(p)
            wfire(t)
        return c

    lax.fori_loop(0, ROWS_W // 2, step_body, 0)
    wwait()


def kernel(x, tok_weight, pos_weight):
    comb = _build_comb(tok_weight, pos_weight)
    comb_flat = comb.reshape(MAX_LEN * NUM_COLORS, D_MODEL)
    x_pad = jnp.pad(x.astype(jnp.int32), ((0, 0), (0, LROW - L)))
    return _sc_gather(comb_flat, x_pad.reshape(-1))


# speedup vs baseline: 1.0027x; 1.0027x over previous
import functools

import jax
import jax.numpy as jnp
from jax import lax
from jax.experimental import pallas as pl
from jax.experimental.pallas import tpu as pltpu
from jax.experimental.pallas import tpu_sc as plsc

NUM_COLORS = 10
D_MODEL = 128
MAX_LEN = 900
B = 1024
L = 900

NUM_WORKERS = 32
ROWS_W = B // NUM_WORKERS
LANES = 16
LROW = 912
NVEC = LROW // LANES


def _comb_body(tok_ref, pos_ref, out_ref):
    out_ref[...] = pos_ref[...][:, None, :] + tok_ref[...][None, :, :]


def _build_comb(tok, pos):
    return pl.pallas_call(
        _comb_body,
        out_shape=jax.ShapeDtypeStruct((MAX_LEN, NUM_COLORS, D_MODEL),
                                       jnp.float32),
    )(tok, pos)


_sc_mesh = plsc.VectorSubcoreMesh(core_axis_name="c", subcore_axis_name="s")


@functools.partial(
    pl.kernel,
    out_type=jax.ShapeDtypeStruct((B, L, D_MODEL), jnp.float32),
    mesh=_sc_mesh,
    scratch_types=[
        pltpu.VMEM((LROW,), jnp.int32),
        pltpu.VMEM((LROW,), jnp.int32),
        pltpu.VMEM((1024,), jnp.int32),
        pltpu.VMEM((1024,), jnp.int32),
        pltpu.VMEM((L, D_MODEL), jnp.float32),
        pltpu.SemaphoreType.DMA,
        pltpu.SemaphoreType.DMA,
    ],
)
def _sc_gather(comb_hbm, x_hbm, out_hbm, xv, l10v, idx0, idx1, rowsv,
               gsem, wsem):
    s = lax.axis_index("s")
    wid = s * 2 + lax.axis_index("c")
    row0 = wid * ROWS_W
    idxb = (idx0, idx1)
    lane = lax.iota(jnp.int32, LANES)

    def l10_body(k, c):
        o = k * LANES
        l10v[pl.ds(o, LANES)] = (o + lane) * NUM_COLORS
        return c

    lax.fori_loop(0, NVEC, l10_body, 0)

    def _row_of(t):
        return row0 + lax.rem(t + 2 * s, jnp.int32(ROWS_W))

    def stage_x(t):
        pltpu.sync_copy(x_hbm.at[pl.ds(_row_of(t) * LROW, LROW)], xv)

    def idx_compute(p):
        def body(k, c):
            o = k * LANES
            idxb[p][pl.ds(o, LANES)] = l10v[pl.ds(o, LANES)] + \
                xv[pl.ds(o, LANES)]
            return c

        lax.fori_loop(0, NVEC, body, 0)

    def gfire(p):
        for j in range(7):
            pltpu.async_copy(
                comb_hbm.at[idxb[p].at[pl.ds(j * 120, 120)]],
                rowsv.at[pl.ds(j * 120, 120)], gsem)
        pltpu.async_copy(
            comb_hbm.at[idxb[p].at[pl.ds(840, 60)]],
            rowsv.at[pl.ds(840, 60)], gsem)

    def gdrain(p):
        for j in range(7):
            pltpu.make_async_copy(
                comb_hbm.at[idxb[p].at[pl.ds(j * 120, 120)]],
                rowsv.at[pl.ds(j * 120, 120)], gsem).wait()
        pltpu.make_async_copy(
            comb_hbm.at[idxb[p].at[pl.ds(840, 60)]],
            rowsv.at[pl.ds(840, 60)], gsem).wait()

    def wfire(t):
        pltpu.async_copy(rowsv, out_hbm.at[_row_of(t)], wsem)

    def wwait():
        pltpu.make_async_copy(rowsv, out_hbm.at[0], wsem).wait()

    stage_x(0)
    idx_compute(0)

    def step_body(k, c):
        for sub in range(2):
            t = k * 2 + sub
            p = sub
            if sub == 0:
                @pl.when(k > 0)
                def _():
                    wwait()
            else:
                wwait()

            gfire(p)

            if sub == 0:
                stage_x(t + 1)
                idx_compute(1 - p)
            else:
                @pl.when(k < ROWS_W // 2 - 1)
                def _(t=t, p=p):
                    stage_x(t + 1)
                    idx_compute(1 - p)

            gdrain(p)
            wfire(t)
        return c

    lax.fori_loop(0, ROWS_W // 2, step_body, 0)
    wwait()


def kernel(x, tok_weight, pos_weight):
    comb = _build_comb(tok_weight, pos_weight)
    comb_flat = comb.reshape(MAX_LEN * NUM_COLORS, D_MODEL)
    x_pad = jnp.pad(x.astype(jnp.int32), ((0, 0), (0, LROW - L)))
    return _sc_gather(comb_flat, x_pad.reshape(-1))
